# baseline (speedup 1.0000x reference)
import functools

import jax
import jax.numpy as jnp
from jax import lax
from jax.experimental import pallas as pl
from jax.experimental.pallas import tpu as pltpu
from jax.experimental.pallas import tpu_sc as plsc

_NLN9 = -2.1972245773362196
_NINTH = 0.1111111111111111
_N = 16777216
_NC, _NS, _L = 2, 16, 16
_NW = _NC * _NS
_PER_W = _N // _NW
_ROWS = 128
_CHUNK = _ROWS * 128
_NCHUNKS = _PER_W // _CHUNK
_NBUF = 2

_mesh = plsc.VectorSubcoreMesh(
    core_axis_name="c", subcore_axis_name="s",
    num_cores=_NC, num_subcores=_NS)


@functools.partial(
    pl.kernel,
    out_type=jax.ShapeDtypeStruct((_N // 128, 128), jnp.float32),
    mesh=_mesh,
    scratch_types=[
        pltpu.VMEM((_NBUF, _ROWS, 128), jnp.float32),
        pltpu.VMEM((_NBUF, _ROWS, 128), jnp.float32),
        pltpu.VMEM((_L,), jnp.float32),
        pltpu.VMEM((_L,), jnp.float32),
    ] + [pltpu.SemaphoreType.DMA] * (2 * _NBUF),
)
def _bspline_sc2(x_hbm, a_hbm, d_hbm, out_hbm, xbuf, ybuf, a_v, d_v,
                 in0, in1, out0, out1):
    insem = (in0, in1)
    outsem = (out0, out1)
    wid = lax.axis_index("s") * _NC + lax.axis_index("c")
    rbase = wid * (_PER_W // 128)
    pltpu.sync_copy(a_hbm, a_v)
    pltpu.sync_copy(d_hbm, d_v)
    av = a_v[...]
    dv = d_v[...]

    def in_slice(c):
        return x_hbm.at[pl.ds(rbase + c * _ROWS, _ROWS)]

    def out_slice(c):
        return out_hbm.at[pl.ds(rbase + c * _ROWS, _ROWS)]

    for b in range(_NBUF):
        pltpu.async_copy(in_slice(b), xbuf.at[b], insem[b])

    @pl.loop(0, _NCHUNKS, step=_NBUF)
    def _outer(c0):
        for b in range(_NBUF):
            c = c0 + b
            pltpu.make_async_copy(in_slice(c), xbuf.at[b], insem[b]).wait()

            @pl.when(c >= _NBUF)
            def _():
                pltpu.make_async_copy(
                    ybuf.at[b], out_slice(c - _NBUF), outsem[b]).wait()

            @plsc.parallel_loop(0, _CHUNK // _L, unroll=8)
            def _vec(i):
                r = i >> 3
                jo = (i & 7) * _L
                x = xbuf[b, r, pl.ds(jo, _L)]
                t = 1.0 / (jnp.exp(_NLN9 - x) + _NINTH)
                k = t.astype(jnp.int32)
                ga = av.at[k].get(mode="promise_in_bounds")
                gd = dv.at[k].get(mode="promise_in_bounds")
                ybuf[b, r, pl.ds(jo, _L)] = ga + gd * t

            @pl.when(c + _NBUF < _NCHUNKS)
            def _():
                pltpu.async_copy(in_slice(c + _NBUF), xbuf.at[b], insem[b])

            pltpu.async_copy(ybuf.at[b], out_slice(c), outsem[b])

    for b in range(_NBUF):
        c = _NCHUNKS - _NBUF + b
        pltpu.make_async_copy(ybuf.at[b], out_slice(c), outsem[b]).wait()


def kernel(x, coeffs):
    coeffs = coeffs.astype(jnp.float32)
    a = jnp.zeros((_L,), jnp.float32).at[:13].set(coeffs)
    d = jnp.zeros((_L,), jnp.float32).at[:12].set(coeffs[1:] - coeffs[:-1])
    a = a - jnp.arange(_L, dtype=jnp.float32) * d
    return _bspline_sc2(x.reshape(_N // 128, 128), a, d).reshape(_N)
